# baseline (speedup 1.0000x reference)
import jax
import jax.numpy as jnp
from jax import lax
from jax.experimental import pallas as pl
from jax.experimental.pallas import tpu as pltpu

M = 1000000
D = 64
B = 16384

CB = 32768
NB = 31
LG_CB = 15
CAP = 1024


def _merge_body(idx_ref, x_ref, u_ref, o_ref, lists, counts, ovfb, ovfc, ovfn):
    s = pl.program_id(0)

    @pl.when(s == 0)
    def _bucketize():
        def zero(w, _):
            counts[w] = 0
            return 0

        lax.fori_loop(0, NB + 1, zero, 0)
        ovfn[0] = 0

        def put(b, _):
            v = idx_ref[b]
            w = v >> LG_CB
            c = counts[w]
            slot = jnp.minimum(c, CAP)
            lists[w, slot] = ((v - (w << LG_CB)) << 14) | b
            oc = ovfn[0]
            spill = c >= CAP
            opos = jnp.where(spill, oc, B)
            ovfb[opos] = b
            ovfc[opos] = v
            ovfn[0] = oc + jnp.where(spill, 1, 0)
            counts[w] = c + 1
            return 0

        lax.fori_loop(0, B, put, 0)

    o_ref[...] = x_ref[...]
    io = lax.broadcasted_iota(jnp.int32, (D, 128), 1)

    def apply(b, local):
        t_off = pl.multiple_of((local >> 7) * 128, 128)
        lane = local & 127
        utile = u_ref[:, b >> 7, :]
        rolled = pltpu.roll(utile, -(b & 127), 1)
        col = lax.broadcast_in_dim(rolled[:, 0:1], (D, 128), (0, 1))
        tgt = o_ref[:, pl.ds(t_off, 128)]
        o_ref[:, pl.ds(t_off, 128)] = jnp.where(io == lane, col, tgt)

    def from_list(j, _):
        e = lists[s, j]
        apply(e & 0x3FFF, e >> 14)
        return 0

    lax.fori_loop(0, jnp.minimum(counts[s], CAP), from_list, 0)

    def from_ovf(j, _):
        v = ovfc[j]

        @pl.when((v >> LG_CB) == s)
        def _():
            apply(ovfb[j], v - (s << LG_CB))

        return 0

    lax.fori_loop(0, ovfn[0], from_ovf, 0)


@jax.jit
def kernel(data, indices, updates):
    idx = indices.reshape(B)
    data_t = data.T
    upd3 = updates.T.reshape(D, 128, 128)

    out_t = pl.pallas_call(
        _merge_body,
        grid_spec=pltpu.PrefetchScalarGridSpec(
            num_scalar_prefetch=1,
            grid=(NB,),
            in_specs=[
                pl.BlockSpec((D, CB), lambda s, idx_ref: (0, s)),
                pl.BlockSpec((D, 128, 128), lambda s, idx_ref: (0, 0, 0)),
            ],
            out_specs=pl.BlockSpec((D, CB), lambda s, idx_ref: (0, s)),
            scratch_shapes=[
                pltpu.SMEM((NB + 1, CAP + 1), jnp.int32),
                pltpu.SMEM((NB + 1,), jnp.int32),
                pltpu.SMEM((B + 1,), jnp.int32),
                pltpu.SMEM((B + 1,), jnp.int32),
                pltpu.SMEM((1,), jnp.int32),
            ],
        ),
        out_shape=jax.ShapeDtypeStruct((D, M), jnp.float32),
    )(idx, data_t, upd3)

    return out_t.T


# speedup vs baseline: 1.1616x; 1.1616x over previous
import jax
import jax.numpy as jnp
from jax import lax
from jax.experimental import pallas as pl
from jax.experimental.pallas import tpu as pltpu

M = 1000000
D = 64
B = 16384

CB = 32768
NB = 31
LG_CB = 15
CAP = 1024


def _merge_body(idx_ref, x_ref, u_ref, o_ref, lists, counts, ovfb, ovfc, ovfn):
    s = pl.program_id(0)

    @pl.when(s == 0)
    def _bucketize():
        def zero(w, _):
            counts[w] = 0
            return 0

        lax.fori_loop(0, NB + 1, zero, 0)
        ovfn[0] = 0

        def put(b, _):
            v = idx_ref[b]
            w = v >> LG_CB
            c = counts[w]
            slot = jnp.minimum(c, CAP)
            lists[w, slot] = ((v - (w << LG_CB)) << 14) | b
            oc = ovfn[0]
            spill = c >= CAP
            opos = jnp.where(spill, oc, B)
            ovfb[opos] = b
            ovfc[opos] = v
            ovfn[0] = oc + jnp.where(spill, 1, 0)
            counts[w] = c + 1
            return 0

        lax.fori_loop(0, B, put, 0)

    o_ref[...] = x_ref[...]
    io = lax.broadcasted_iota(jnp.int32, (D, 128), 1)

    def apply(b, local):
        t_off = pl.multiple_of((local >> 7) * 128, 128)
        lane = local & 127
        utile = u_ref[b >> 7]
        rolled = pltpu.roll(utile, -(b & 127), 1)
        col = lax.broadcast_in_dim(rolled[:, 0:1], (D, 128), (0, 1))
        tgt = o_ref[:, pl.ds(t_off, 128)]
        o_ref[:, pl.ds(t_off, 128)] = jnp.where(io == lane, col, tgt)

    def from_list(j, _):
        e = lists[s, j]
        apply(e & 0x3FFF, e >> 14)
        return 0

    lax.fori_loop(0, jnp.minimum(counts[s], CAP), from_list, 0)

    def from_ovf(j, _):
        v = ovfc[j]

        @pl.when((v >> LG_CB) == s)
        def _():
            apply(ovfb[j], v - (s << LG_CB))

        return 0

    lax.fori_loop(0, ovfn[0], from_ovf, 0)


@jax.jit
def kernel(data, indices, updates):
    idx = indices.reshape(B)
    data_t = data.T
    upd3 = jnp.transpose(updates.reshape(128, 128, D), (0, 2, 1))

    out_t = pl.pallas_call(
        _merge_body,
        grid_spec=pltpu.PrefetchScalarGridSpec(
            num_scalar_prefetch=1,
            grid=(NB,),
            in_specs=[
                pl.BlockSpec((D, CB), lambda s, idx_ref: (0, s)),
                pl.BlockSpec((128, D, 128), lambda s, idx_ref: (0, 0, 0)),
            ],
            out_specs=pl.BlockSpec((D, CB), lambda s, idx_ref: (0, s)),
            scratch_shapes=[
                pltpu.SMEM((NB + 1, CAP + 1), jnp.int32),
                pltpu.SMEM((NB + 1,), jnp.int32),
                pltpu.SMEM((B + 1,), jnp.int32),
                pltpu.SMEM((B + 1,), jnp.int32),
                pltpu.SMEM((1,), jnp.int32),
            ],
        ),
        out_shape=jax.ShapeDtypeStruct((D, M), jnp.float32),
    )(idx, data_t, upd3)

    return out_t.T
